# Initial kernel scaffold; baseline (speedup 1.0000x reference)
import jax
import jax.numpy as jnp
from jax.experimental import pallas as pl


def kernel(node_emb, edge_emb, i, W1, b1, gamma1, beta1, gamma2, beta2):
    raise NotImplementedError("write your pallas kernel here")


# speedup vs baseline: 2.0352x; 2.0352x over previous
import functools

import jax
import jax.numpy as jnp
from jax import lax
from jax.experimental import pallas as pl
from jax.experimental.pallas import tpu as pltpu
from jax.experimental.pallas import tpu_sc as plsc


def _node_mm_body(node_ref, wn_ref, p_ref):
    p_ref[...] = lax.dot_general(
        node_ref[...], wn_ref[...], (((1,), (1,)), ((), ())),
        preferred_element_type=jnp.float32)


def _edge_mm_stats_body(edge_ref, g_ref, we_ref, prm_ref, c1_ref, st_ref):
    c1 = lax.dot_general(
        edge_ref[...], we_ref[...], (((1,), (1,)), ((), ())),
        preferred_element_type=jnp.float32)
    c1 = c1 + g_ref[...] + prm_ref[0:1, :]
    c1_ref[...] = c1

    @pl.when(pl.program_id(0) == 0)
    def _():
        st_ref[...] = jnp.zeros_like(st_ref)

    s = jnp.sum(c1, axis=0, keepdims=True)
    q = jnp.sum(c1 * c1, axis=0, keepdims=True)
    pad = jnp.zeros((st_ref.shape[0] - 2, c1.shape[1]), jnp.float32)
    st_ref[...] += jnp.concatenate([s, q, pad], axis=0)


def _act_body(c1_ref, st_ref, prm_ref, msg_ref, *, inv_e, hn):
    mu = st_ref[0:1, :] * inv_e
    var = st_ref[1:2, :] * inv_e - mu * mu
    scale = prm_ref[1:2, :] * lax.rsqrt(var + 1e-5)
    shift = prm_ref[2:3, :] - mu * scale
    y = c1_ref[...] * scale + shift
    f = y[:, :hn]
    c = y[:, hn:]
    msg_ref[...] = jax.nn.sigmoid(f) * jnp.tanh(c)


def _final_body(pa_ref, node_ref, prm_ref, out_ref, *, inv_n):
    a = pa_ref[0] + pa_ref[1]
    mu = jnp.sum(a, axis=0, keepdims=True) * inv_n
    d = a - mu
    var = jnp.sum(d * d, axis=0, keepdims=True) * inv_n
    bn = d * lax.rsqrt(var + 1e-5) * prm_ref[0:1, :] + prm_ref[1:2, :]
    out_ref[...] = jnp.tanh(node_ref[...] + bn)


def kernel(node_emb, edge_emb, i, W1, b1, gamma1, beta1, gamma2, beta2):
    N, HN = node_emb.shape
    E, HE = edge_emb.shape
    H2 = W1.shape[0]

    Wn = W1[:, :HN]
    We = W1[:, HN:]
    prm1 = jnp.concatenate(
        [b1[None], gamma1[None], beta1[None],
         jnp.zeros((5, H2), jnp.float32)], axis=0)
    prm2 = jnp.concatenate(
        [gamma2[None], beta2[None], jnp.zeros((6, HN), jnp.float32)], axis=0)

    P = pl.pallas_call(
        _node_mm_body,
        out_shape=jax.ShapeDtypeStruct((N, H2), jnp.float32),
    )(node_emb, Wn)

    mesh = plsc.VectorSubcoreMesh(core_axis_name="core",
                                  subcore_axis_name="subcore")
    WIN = 128
    SC_TILES = 32
    E_pad = ((E + WIN * SC_TILES - 1) // (WIN * SC_TILES)) * (WIN * SC_TILES)
    i_pad = jnp.pad(i, (0, E_pad - E)).reshape(1, E_pad)

    @functools.partial(
        pl.kernel,
        out_type=jax.ShapeDtypeStruct((E_pad, H2), jnp.float32),
        mesh=mesh)
    def _gather(p_hbm, i_hbm, g_hbm):
        def body(i_vmem, o_vmem):
            pltpu.sync_copy(p_hbm.at[i_vmem.at[0]], o_vmem)

        pltpu.emit_pipeline(
            body,
            grid=(E_pad // WIN,),
            in_specs=[pl.BlockSpec((1, WIN), index_map=lambda k: (0, k))],
            out_specs=[pl.BlockSpec((WIN, H2), index_map=lambda k: (k, 0))],
            core_axis_name=("core", "subcore"),
            dimension_semantics=(pltpu.PARALLEL,),
        )(i_hbm, g_hbm)

    G = _gather(P, i_pad)

    TILE = 2000
    grid_e = E // TILE
    c1, stats = pl.pallas_call(
        _edge_mm_stats_body,
        grid=(grid_e,),
        in_specs=[
            pl.BlockSpec((TILE, HE), lambda t: (t, 0)),
            pl.BlockSpec((TILE, H2), lambda t: (t, 0)),
            pl.BlockSpec((H2, HE), lambda t: (0, 0)),
            pl.BlockSpec((8, H2), lambda t: (0, 0)),
        ],
        out_specs=[
            pl.BlockSpec((TILE, H2), lambda t: (t, 0)),
            pl.BlockSpec((8, H2), lambda t: (0, 0)),
        ],
        out_shape=[
            jax.ShapeDtypeStruct((E, H2), jnp.float32),
            jax.ShapeDtypeStruct((8, H2), jnp.float32),
        ],
    )(edge_emb, G, We, prm1)

    msg = pl.pallas_call(
        functools.partial(_act_body, inv_e=1.0 / E, hn=HN),
        grid=(grid_e,),
        in_specs=[
            pl.BlockSpec((TILE, H2), lambda t: (t, 0)),
            pl.BlockSpec((8, H2), lambda t: (0, 0)),
            pl.BlockSpec((8, H2), lambda t: (0, 0)),
        ],
        out_specs=pl.BlockSpec((TILE, HN), lambda t: (t, 0)),
        out_shape=jax.ShapeDtypeStruct((E, HN), jnp.float32),
    )(c1, stats, prm1)

    CH = 80
    EC = E // 2
    RT = EC // 16
    n_chunks = RT // CH
    zeros_init = jnp.zeros((N, HN), jnp.float32)

    @functools.partial(
        pl.kernel,
        out_type=jax.ShapeDtypeStruct((2, N, HN), jnp.float32),
        mesh=mesh,
        scratch_types=[
            pltpu.VMEM((CH, HN), jnp.float32),
            pltpu.VMEM((CH,), jnp.int32),
            pltpu.VMEM_SHARED((N, HN), jnp.float32),
        ])
    def _scatter(msg_hbm, i_hbm, zero_hbm, out_hbm, msg_v, idx_v, acc):
        cid = lax.axis_index("core")
        sid = lax.axis_index("subcore")

        @pl.when(sid == 0)
        def _():
            pltpu.sync_copy(zero_hbm, acc)

        plsc.subcore_barrier()
        base0 = cid * EC + sid * RT

        @pl.loop(0, n_chunks)
        def _(k):
            b = base0 + k * CH
            pltpu.sync_copy(i_hbm.at[pl.ds(b, CH)], idx_v)
            pltpu.sync_copy(msg_hbm.at[pl.ds(b, CH), :], msg_v)
            pltpu.sync_copy(msg_v, acc.at[idx_v], add=True)

        plsc.subcore_barrier()

        @pl.when(sid == 0)
        def _():
            pltpu.sync_copy(acc, out_hbm.at[cid])

    partials = _scatter(msg, i, zeros_init)

    out = pl.pallas_call(
        functools.partial(_final_body, inv_n=1.0 / N),
        out_shape=jax.ShapeDtypeStruct((N, HN), jnp.float32),
    )(partials, node_emb, prm2)
    return out


# speedup vs baseline: 2.2959x; 1.1281x over previous
import functools

import jax
import jax.numpy as jnp
from jax import lax
from jax.experimental import pallas as pl
from jax.experimental.pallas import tpu as pltpu
from jax.experimental.pallas import tpu_sc as plsc


def _node_mm_body(node_ref, wn_ref, p_ref):
    p = lax.dot_general(
        node_ref[...], wn_ref[...], (((1,), (1,)), ((), ())),
        preferred_element_type=jnp.float32)
    hn = p.shape[1] // 2
    lo = lax.bitcast_convert_type(p[:, :hn], jnp.int32)
    hi = lax.bitcast_convert_type(p[:, hn:], jnp.int32)
    lo = lax.shift_right_logical(lo + 0x8000, 16)
    hi = (hi + 0x8000) & jnp.int32(0xFFFF0000 - (1 << 32))
    p_ref[...] = lo | hi


def _unpack_pair(u):
    f = lax.bitcast_convert_type(lax.shift_left(u, 16), jnp.float32)
    c = lax.bitcast_convert_type(u & jnp.int32(0xFFFF0000 - (1 << 32)),
                                 jnp.float32)
    return f, c


def _edge_mm_stats_body(edge_ref, g_ref, we_ref, prm_ref, c1_ref, st_ref):
    q = lax.dot_general(
        edge_ref[...], we_ref[...], (((1,), (1,)), ((), ())),
        preferred_element_type=jnp.float32)
    hn = q.shape[1] // 2
    gf, gc = _unpack_pair(g_ref[...])
    c1 = jnp.concatenate([q[:, :hn] + gf, q[:, hn:] + gc], axis=1)
    c1 = c1 + prm_ref[0:1, :]
    c1_ref[...] = c1

    @pl.when(pl.program_id(0) == 0)
    def _():
        st_ref[...] = jnp.zeros_like(st_ref)

    s = jnp.sum(c1, axis=0, keepdims=True)
    q = jnp.sum(c1 * c1, axis=0, keepdims=True)
    pad = jnp.zeros((st_ref.shape[0] - 2, c1.shape[1]), jnp.float32)
    st_ref[...] += jnp.concatenate([s, q, pad], axis=0)


def _act_body(c1_ref, st_ref, prm_ref, msg_ref, *, inv_e, hn):
    mu = st_ref[0:1, :] * inv_e
    var = st_ref[1:2, :] * inv_e - mu * mu
    scale = prm_ref[1:2, :] * lax.rsqrt(var + 1e-5)
    shift = prm_ref[2:3, :] - mu * scale
    y = c1_ref[...] * scale + shift
    f = y[:, :hn]
    c = y[:, hn:]
    msg_ref[...] = jax.nn.sigmoid(f) * jnp.tanh(c)


def _final_body(pa_ref, node_ref, prm_ref, out_ref, *, inv_n):
    a = pa_ref[0] + pa_ref[1]
    mu = jnp.sum(a, axis=0, keepdims=True) * inv_n
    d = a - mu
    var = jnp.sum(d * d, axis=0, keepdims=True) * inv_n
    bn = d * lax.rsqrt(var + 1e-5) * prm_ref[0:1, :] + prm_ref[1:2, :]
    out_ref[...] = jnp.tanh(node_ref[...] + bn)


def kernel(node_emb, edge_emb, i, W1, b1, gamma1, beta1, gamma2, beta2):
    N, HN = node_emb.shape
    E, HE = edge_emb.shape
    H2 = W1.shape[0]

    Wn = W1[:, :HN]
    We = W1[:, HN:]
    prm1 = jnp.concatenate(
        [b1[None], gamma1[None], beta1[None],
         jnp.zeros((5, H2), jnp.float32)], axis=0)
    prm2 = jnp.concatenate(
        [gamma2[None], beta2[None], jnp.zeros((6, HN), jnp.float32)], axis=0)

    P = pl.pallas_call(
        _node_mm_body,
        out_shape=jax.ShapeDtypeStruct((N, HN), jnp.int32),
    )(node_emb, Wn)

    mesh = plsc.VectorSubcoreMesh(core_axis_name="core",
                                  subcore_axis_name="subcore")
    WIN = 128
    SC_TILES = 32
    E_pad = ((E + WIN * SC_TILES - 1) // (WIN * SC_TILES)) * (WIN * SC_TILES)
    i_pad = jnp.pad(i, (0, E_pad - E)).reshape(1, E_pad)

    @functools.partial(
        pl.kernel,
        out_type=jax.ShapeDtypeStruct((E_pad, HN), jnp.int32),
        mesh=mesh)
    def _gather(p_hbm, i_hbm, g_hbm):
        def body(i_vmem, o_vmem):
            pltpu.sync_copy(p_hbm.at[i_vmem.at[0]], o_vmem)

        pltpu.emit_pipeline(
            body,
            grid=(E_pad // WIN,),
            in_specs=[pl.BlockSpec((1, WIN), index_map=lambda k: (0, k))],
            out_specs=[pl.BlockSpec((WIN, HN), index_map=lambda k: (k, 0))],
            core_axis_name=("core", "subcore"),
            dimension_semantics=(pltpu.PARALLEL,),
        )(i_hbm, g_hbm)

    G = _gather(P, i_pad)

    TILE = 2000
    grid_e = E // TILE
    c1, stats = pl.pallas_call(
        _edge_mm_stats_body,
        grid=(grid_e,),
        in_specs=[
            pl.BlockSpec((TILE, HE), lambda t: (t, 0)),
            pl.BlockSpec((TILE, HN), lambda t: (t, 0)),
            pl.BlockSpec((H2, HE), lambda t: (0, 0)),
            pl.BlockSpec((8, H2), lambda t: (0, 0)),
        ],
        out_specs=[
            pl.BlockSpec((TILE, H2), lambda t: (t, 0)),
            pl.BlockSpec((8, H2), lambda t: (0, 0)),
        ],
        out_shape=[
            jax.ShapeDtypeStruct((E, H2), jnp.float32),
            jax.ShapeDtypeStruct((8, H2), jnp.float32),
        ],
    )(edge_emb, G, We, prm1)

    msg = pl.pallas_call(
        functools.partial(_act_body, inv_e=1.0 / E, hn=HN),
        grid=(grid_e,),
        in_specs=[
            pl.BlockSpec((TILE, H2), lambda t: (t, 0)),
            pl.BlockSpec((8, H2), lambda t: (0, 0)),
            pl.BlockSpec((8, H2), lambda t: (0, 0)),
        ],
        out_specs=pl.BlockSpec((TILE, HN), lambda t: (t, 0)),
        out_shape=jax.ShapeDtypeStruct((E, HN), jnp.float32),
    )(c1, stats, prm1)

    CH = 80
    EC = E // 2
    RT = EC // 16
    n_chunks = RT // CH
    zeros_init = jnp.zeros((N, HN), jnp.float32)

    @functools.partial(
        pl.kernel,
        out_type=jax.ShapeDtypeStruct((2, N, HN), jnp.float32),
        mesh=mesh,
        scratch_types=[
            pltpu.VMEM((CH, HN), jnp.float32),
            pltpu.VMEM((CH,), jnp.int32),
            pltpu.VMEM_SHARED((N, HN), jnp.float32),
        ])
    def _scatter(msg_hbm, i_hbm, zero_hbm, out_hbm, msg_v, idx_v, acc):
        cid = lax.axis_index("core")
        sid = lax.axis_index("subcore")

        @pl.when(sid == 0)
        def _():
            pltpu.sync_copy(zero_hbm, acc)

        plsc.subcore_barrier()
        base0 = cid * EC + sid * RT

        @pl.loop(0, n_chunks)
        def _(k):
            b = base0 + k * CH
            pltpu.sync_copy(i_hbm.at[pl.ds(b, CH)], idx_v)
            pltpu.sync_copy(msg_hbm.at[pl.ds(b, CH), :], msg_v)
            pltpu.sync_copy(msg_v, acc.at[idx_v], add=True)

        plsc.subcore_barrier()

        @pl.when(sid == 0)
        def _():
            pltpu.sync_copy(acc, out_hbm.at[cid])

    partials = _scatter(msg, i, zeros_init)

    out = pl.pallas_call(
        functools.partial(_final_body, inv_n=1.0 / N),
        out_shape=jax.ShapeDtypeStruct((N, HN), jnp.float32),
    )(partials, node_emb, prm2)
    return out


# speedup vs baseline: 3.9102x; 1.7031x over previous
import functools

import jax
import jax.numpy as jnp
from jax import lax
from jax.experimental import pallas as pl
from jax.experimental.pallas import tpu as pltpu
from jax.experimental.pallas import tpu_sc as plsc


SC_TILES = 32
CH = 128
_HI_MASK = -65536


def _pack_pair(a, b):
    lo = lax.shift_right_logical(
        lax.bitcast_convert_type(a, jnp.int32) + 0x8000, 16)
    hi = (lax.bitcast_convert_type(b, jnp.int32) + 0x8000) & _HI_MASK
    return lo | hi


def _unpack_pair(u):
    f = lax.bitcast_convert_type(lax.shift_left(u, 16), jnp.float32)
    c = lax.bitcast_convert_type(u & _HI_MASK, jnp.float32)
    return f, c


def _node_mm_body(node_ref, wn_ref, p_ref):
    p = lax.dot_general(
        node_ref[...], wn_ref[...], (((1,), (1,)), ((), ())),
        preferred_element_type=jnp.float32)
    hn = p.shape[1] // 2
    p_ref[...] = _pack_pair(p[:, :hn], p[:, hn:])


def _edge_mm_stats_body(edge_ref, g_ref, we_ref, prm_ref, c1_ref, st_ref):
    q = lax.dot_general(
        edge_ref[...].astype(jnp.bfloat16), we_ref[...].astype(jnp.bfloat16),
        (((1,), (1,)), ((), ())), preferred_element_type=jnp.float32)
    hn = q.shape[1] // 2
    gf, gc = _unpack_pair(g_ref[...])
    c1f = q[:, :hn] + gf + prm_ref[0:1, :hn]
    c1c = q[:, hn:] + gc + prm_ref[0:1, hn:]
    c1_ref[...] = _pack_pair(c1f, c1c)

    @pl.when(pl.program_id(0) == 0)
    def _():
        st_ref[...] = jnp.zeros_like(st_ref)

    c1 = jnp.concatenate([c1f, c1c], axis=1)
    s = jnp.sum(c1, axis=0, keepdims=True)
    q2 = jnp.sum(c1 * c1, axis=0, keepdims=True)
    pad = jnp.zeros((st_ref.shape[0] - 2, c1.shape[1]), jnp.float32)
    st_ref[...] += jnp.concatenate([s, q2, pad], axis=0)


def _act_body(c1_ref, sta_ref, stb_ref, prm_ref, msg_ref, *, inv_e, hn):
    st = sta_ref[...] + stb_ref[...]
    mu = st[0:1, :] * inv_e
    var = st[1:2, :] * inv_e - mu * mu
    scale = prm_ref[1:2, :] * lax.rsqrt(var + 1e-5)
    shift = prm_ref[2:3, :] - mu * scale
    c1f, c1c = _unpack_pair(c1_ref[...])
    f = c1f * scale[:, :hn] + shift[:, :hn]
    c = c1c * scale[:, hn:] + shift[:, hn:]
    msg_ref[...] = jax.nn.sigmoid(f) * jnp.tanh(c)


def _final_body(pa_ref, pb_ref, node_ref, prm_ref, out_ref, *, inv_n):
    a = (pa_ref[0] + pa_ref[1]) + (pb_ref[0] + pb_ref[1])
    mu = jnp.sum(a, axis=0, keepdims=True) * inv_n
    d = a - mu
    var = jnp.sum(d * d, axis=0, keepdims=True) * inv_n
    bn = d * lax.rsqrt(var + 1e-5) * prm_ref[0:1, :] + prm_ref[1:2, :]
    out_ref[...] = jnp.tanh(node_ref[...] + bn)


def _sc_chunking(n_rows):
    rt = n_rows // SC_TILES
    full = rt // CH
    rem = full % 2
    tail = rt - full * CH
    return rt, full, rem, tail


def _make_gather(mesh, HN, base_rows, n_rows):
    RT, full, rem, TAIL = _sc_chunking(n_rows)
    MAIN = full - rem

    @functools.partial(
        pl.kernel,
        out_type=jax.ShapeDtypeStruct((n_rows, HN), jnp.int32),
        mesh=mesh,
        scratch_types=[
            pltpu.VMEM((CH,), jnp.int32), pltpu.VMEM((CH,), jnp.int32),
            pltpu.VMEM((CH, HN), jnp.int32), pltpu.VMEM((CH, HN), jnp.int32),
            pltpu.SemaphoreType.DMA, pltpu.SemaphoreType.DMA,
            pltpu.SemaphoreType.DMA, pltpu.SemaphoreType.DMA,
        ])
    def _gather(p_hbm, i_hbm, g_hbm,
                ib0, ib1, rb0, rb1, si0, si1, sw0, sw1):
        cid = lax.axis_index("core")
        sid = lax.axis_index("subcore")
        wid = cid * 16 + sid
        base_i = base_rows + wid * RT
        base_o = wid * RT

        def idx_copy(k, ib, si):
            return pltpu.make_async_copy(
                i_hbm.at[pl.ds(base_i + k * CH, CH)], ib, si)

        def out_copy(k, rb, sw):
            return pltpu.make_async_copy(
                rb, g_hbm.at[pl.ds(base_o + k * CH, CH), :], sw)

        idx_copy(0, ib0, si0).start()
        idx_copy(1, ib1, si1).start()
        out_copy(0, rb0, sw0).start()
        out_copy(1, rb1, sw1).start()

        @pl.loop(0, MAIN, step=2)
        def _(g):
            for b, (ib, rb, si, sw) in enumerate(
                    ((ib0, rb0, si0, sw0), (ib1, rb1, si1, sw1))):
                k = g + b
                idx_copy(k, ib, si).wait()
                out_copy(k, rb, sw).wait()
                pltpu.sync_copy(p_hbm.at[ib], rb)
                kn = jnp.minimum(k + 2, full - 1)
                idx_copy(kn, ib, si).start()
                out_copy(k, rb, sw).start()

        if rem:
            k = full - 1
            idx_copy(k, ib0, si0).wait()
            out_copy(k, rb0, sw0).wait()
            pltpu.sync_copy(p_hbm.at[ib0], rb0)
            pltpu.sync_copy(rb0, g_hbm.at[pl.ds(base_o + k * CH, CH), :])
            idx_copy(k, ib1, si1).wait()
            out_copy(k, rb1, sw1).wait()
        else:
            idx_copy(full - 1, ib0, si0).wait()
            idx_copy(full - 1, ib1, si1).wait()
            out_copy(full - 1, rb0, sw0).wait()
            out_copy(full - 1, rb1, sw1).wait()

        if TAIL:
            tb_i = base_i + full * CH
            tb_o = base_o + full * CH
            pltpu.sync_copy(i_hbm.at[pl.ds(tb_i, TAIL)],
                            ib0.at[pl.ds(0, TAIL)])
            pltpu.sync_copy(p_hbm.at[ib0.at[pl.ds(0, TAIL)]],
                            rb0.at[pl.ds(0, TAIL), :])
            pltpu.sync_copy(rb0.at[pl.ds(0, TAIL), :],
                            g_hbm.at[pl.ds(tb_o, TAIL), :])

    return _gather


def _make_scatter(mesh, N, HN, base_rows, n_rows):
    RT, full, rem, TAIL = _sc_chunking(n_rows)
    MAIN = full - rem

    @functools.partial(
        pl.kernel,
        out_type=jax.ShapeDtypeStruct((2, N, HN), jnp.float32),
        mesh=mesh,
        scratch_types=[
            pltpu.VMEM((CH,), jnp.int32), pltpu.VMEM((CH,), jnp.int32),
            pltpu.VMEM((CH, HN), jnp.float32), pltpu.VMEM((CH, HN), jnp.float32),
            pltpu.SemaphoreType.DMA, pltpu.SemaphoreType.DMA,
            pltpu.VMEM((max(8, CH // 16),), jnp.int32),
            pltpu.VMEM_SHARED((N, HN), jnp.float32),
        ])
    def _scatter(msg_hbm, i_hbm, zero_hbm, out_hbm,
                 ib0, ib1, mb0, mb1, s0, s1, it_v, acc):
        cid = lax.axis_index("core")
        sid = lax.axis_index("subcore")
        wid = cid * 16 + sid
        base_i = base_rows + wid * RT
        base_m = wid * RT

        @pl.when(sid == 0)
        def _():
            pltpu.sync_copy(zero_hbm, acc)

        plsc.subcore_barrier()

        def in_copies(k, ib, mb, s):
            return (pltpu.make_async_copy(
                        i_hbm.at[pl.ds(base_i + k * CH, CH)], ib, s),
                    pltpu.make_async_copy(
                        msg_hbm.at[pl.ds(base_m + k * CH, CH), :], mb, s))

        for c in in_copies(0, ib0, mb0, s0) + in_copies(1, ib1, mb1, s1):
            c.start()

        @pl.loop(0, MAIN, step=2)
        def _(g):
            for b, (ib, mb, s) in enumerate(((ib0, mb0, s0), (ib1, mb1, s1))):
                k = g + b
                ca, cb = in_copies(k, ib, mb, s)
                ca.wait()
                cb.wait()
                pltpu.sync_copy(mb, acc.at[ib], add=True)
                kn = jnp.minimum(k + 2, full - 1)
                for c in in_copies(kn, ib, mb, s):
                    c.start()

        if rem:
            k = full - 1
            ca, cb = in_copies(k, ib0, mb0, s0)
            ca.wait()
            cb.wait()
            pltpu.sync_copy(mb0, acc.at[ib0], add=True)
            for c in in_copies(k, ib1, mb1, s1):
                c.wait()
        else:
            for c in in_copies(full - 1, ib0, mb0, s0) + in_copies(
                    full - 1, ib1, mb1, s1):
                c.wait()

        if TAIL:
            tb_i = base_i + full * CH
            tb_m = base_m + full * CH
            pltpu.sync_copy(i_hbm.at[pl.ds(tb_i, TAIL)],
                            it_v.at[pl.ds(0, TAIL)])
            pltpu.sync_copy(msg_hbm.at[pl.ds(tb_m, TAIL), :],
                            mb0.at[pl.ds(0, TAIL), :])
            pltpu.sync_copy(mb0.at[pl.ds(0, TAIL), :],
                            acc.at[it_v.at[pl.ds(0, TAIL)]], add=True)

        plsc.subcore_barrier()

        @pl.when(sid == 0)
        def _():
            pltpu.sync_copy(acc, out_hbm.at[cid])

    return _scatter


def kernel(node_emb, edge_emb, i, W1, b1, gamma1, beta1, gamma2, beta2):
    N, HN = node_emb.shape
    E, HE = edge_emb.shape
    H2 = W1.shape[0]

    Wn = W1[:, :HN]
    We = W1[:, HN:]
    prm1 = jnp.concatenate(
        [b1[None], gamma1[None], beta1[None],
         jnp.zeros((5, H2), jnp.float32)], axis=0)
    prm2 = jnp.concatenate(
        [gamma2[None], beta2[None], jnp.zeros((6, HN), jnp.float32)], axis=0)

    mesh = plsc.VectorSubcoreMesh(core_axis_name="core",
                                  subcore_axis_name="subcore")

    NS = 2
    ES = E // NS
    TILE = 2000
    grid_s = ES // TILE
    zeros_init = jnp.zeros((N, HN), jnp.float32)

    P = pl.pallas_call(
        _node_mm_body,
        out_shape=jax.ShapeDtypeStruct((N, HN), jnp.int32),
    )(node_emb, Wn)

    G = [_make_gather(mesh, HN, s * ES, ES)(P, i) for s in range(NS)]

    c1, stats = [], []
    for s in range(NS):
        c1_s, st_s = pl.pallas_call(
            _edge_mm_stats_body,
            grid=(grid_s,),
            in_specs=[
                pl.BlockSpec((TILE, HE), lambda t, s=s: (t + s * grid_s, 0)),
                pl.BlockSpec((TILE, HN), lambda t: (t, 0)),
                pl.BlockSpec((H2, HE), lambda t: (0, 0)),
                pl.BlockSpec((8, H2), lambda t: (0, 0)),
            ],
            out_specs=[
                pl.BlockSpec((TILE, HN), lambda t: (t, 0)),
                pl.BlockSpec((8, H2), lambda t: (0, 0)),
            ],
            out_shape=[
                jax.ShapeDtypeStruct((ES, HN), jnp.int32),
                jax.ShapeDtypeStruct((8, H2), jnp.float32),
            ],
        )(edge_emb, G[s], We, prm1)
        c1.append(c1_s)
        stats.append(st_s)

    partials = []
    for s in range(NS):
        msg_s = pl.pallas_call(
            functools.partial(_act_body, inv_e=1.0 / E, hn=HN),
            grid=(grid_s,),
            in_specs=[
                pl.BlockSpec((TILE, HN), lambda t: (t, 0)),
                pl.BlockSpec((8, H2), lambda t: (0, 0)),
                pl.BlockSpec((8, H2), lambda t: (0, 0)),
                pl.BlockSpec((8, H2), lambda t: (0, 0)),
            ],
            out_specs=pl.BlockSpec((TILE, HN), lambda t: (t, 0)),
            out_shape=jax.ShapeDtypeStruct((ES, HN), jnp.float32),
        )(c1[s], stats[0], stats[1], prm1)
        partials.append(
            _make_scatter(mesh, N, HN, s * ES, ES)(msg_s, i, zeros_init))

    out = pl.pallas_call(
        functools.partial(_final_body, inv_n=1.0 / N),
        out_shape=jax.ShapeDtypeStruct((N, HN), jnp.float32),
    )(partials[0], partials[1], node_emb, prm2)
    return out


# speedup vs baseline: 4.4492x; 1.1379x over previous
import functools

import jax
import jax.numpy as jnp
from jax import lax
from jax.experimental import pallas as pl
from jax.experimental.pallas import tpu as pltpu
from jax.experimental.pallas import tpu_sc as plsc


SC_TILES = 32
CH = 128
_HI_MASK = -65536


def _pack_pair(a, b):
    lo = lax.shift_right_logical(
        lax.bitcast_convert_type(a, jnp.int32) + 0x8000, 16)
    hi = (lax.bitcast_convert_type(b, jnp.int32) + 0x8000) & _HI_MASK
    return lo | hi


def _unpack_pair(u):
    f = lax.bitcast_convert_type(lax.shift_left(u, 16), jnp.float32)
    c = lax.bitcast_convert_type(u & _HI_MASK, jnp.float32)
    return f, c


def _node_mm_body(node_ref, wn_ref, p_ref):
    p = lax.dot_general(
        node_ref[...], wn_ref[...], (((1,), (1,)), ((), ())),
        preferred_element_type=jnp.float32)
    hn = p.shape[1] // 2
    p_ref[...] = _pack_pair(p[:, :hn], p[:, hn:])


def _edge_mm_stats_body(edge_ref, g_ref, we_ref, prm_ref, c1_ref, st_ref):
    q = lax.dot_general(
        edge_ref[...].astype(jnp.bfloat16), we_ref[...].astype(jnp.bfloat16),
        (((1,), (1,)), ((), ())), preferred_element_type=jnp.float32)
    hn = q.shape[1] // 2
    gf, gc = _unpack_pair(g_ref[...])
    c1f = q[:, :hn] + gf + prm_ref[0:1, :hn]
    c1c = q[:, hn:] + gc + prm_ref[0:1, hn:]
    c1_ref[...] = _pack_pair(c1f, c1c)

    @pl.when(pl.program_id(0) == 0)
    def _():
        st_ref[...] = jnp.zeros_like(st_ref)

    c1 = jnp.concatenate([c1f, c1c], axis=1)
    s = jnp.sum(c1, axis=0, keepdims=True)
    q2 = jnp.sum(c1 * c1, axis=0, keepdims=True)
    pad = jnp.zeros((st_ref.shape[0] - 2, c1.shape[1]), jnp.float32)
    st_ref[...] += jnp.concatenate([s, q2, pad], axis=0)


def _act_body(c1_ref, sta_ref, stb_ref, prm_ref, msg_ref, *, inv_e, hn):
    st = sta_ref[...] + stb_ref[...]
    mu = st[0:1, :] * inv_e
    var = st[1:2, :] * inv_e - mu * mu
    scale = prm_ref[1:2, :] * lax.rsqrt(var + 1e-5)
    shift = prm_ref[2:3, :] - mu * scale
    c1f, c1c = _unpack_pair(c1_ref[...])
    f = c1f * scale[:, :hn] + shift[:, :hn]
    c = c1c * scale[:, hn:] + shift[:, hn:]
    msg_ref[...] = jax.nn.sigmoid(f) * jnp.tanh(c)


def _final_body(pa_ref, pb_ref, node_ref, prm_ref, out_ref, *, inv_n):
    a = (pa_ref[0] + pa_ref[1]) + (pb_ref[0] + pb_ref[1])
    mu = jnp.sum(a, axis=0, keepdims=True) * inv_n
    d = a - mu
    var = jnp.sum(d * d, axis=0, keepdims=True) * inv_n
    bn = d * lax.rsqrt(var + 1e-5) * prm_ref[0:1, :] + prm_ref[1:2, :]
    out_ref[...] = jnp.tanh(node_ref[...] + bn)


def _sc_chunking(n_rows):
    rt = n_rows // SC_TILES
    full = rt // CH
    rem = full % 2
    tail = rt - full * CH
    return rt, full, rem, tail


def _make_gather(mesh, HN, base_rows, n_rows):
    RT, full, rem, TAIL = _sc_chunking(n_rows)
    MAIN = full - rem

    @functools.partial(
        pl.kernel,
        out_type=jax.ShapeDtypeStruct((n_rows, HN), jnp.int32),
        mesh=mesh,
        scratch_types=[
            pltpu.VMEM((CH,), jnp.int32), pltpu.VMEM((CH,), jnp.int32),
            pltpu.VMEM((CH, HN), jnp.int32), pltpu.VMEM((CH, HN), jnp.int32),
            pltpu.SemaphoreType.DMA, pltpu.SemaphoreType.DMA,
            pltpu.SemaphoreType.DMA, pltpu.SemaphoreType.DMA,
        ])
    def _gather(p_hbm, i_hbm, g_hbm,
                ib0, ib1, rb0, rb1, si0, si1, sw0, sw1):
        cid = lax.axis_index("core")
        sid = lax.axis_index("subcore")
        wid = cid * 16 + sid
        base_i = base_rows + wid * RT
        base_o = wid * RT

        def idx_copy(k, ib, si):
            return pltpu.make_async_copy(
                i_hbm.at[pl.ds(base_i + k * CH, CH)], ib, si)

        def out_copy(k, rb, sw):
            return pltpu.make_async_copy(
                rb, g_hbm.at[pl.ds(base_o + k * CH, CH), :], sw)

        idx_copy(0, ib0, si0).start()
        idx_copy(1, ib1, si1).start()
        out_copy(0, rb0, sw0).start()
        out_copy(1, rb1, sw1).start()

        @pl.loop(0, MAIN, step=2)
        def _(g):
            for b, (ib, rb, si, sw) in enumerate(
                    ((ib0, rb0, si0, sw0), (ib1, rb1, si1, sw1))):
                k = g + b
                idx_copy(k, ib, si).wait()
                out_copy(k, rb, sw).wait()
                pltpu.sync_copy(p_hbm.at[ib], rb)
                kn = jnp.minimum(k + 2, full - 1)
                idx_copy(kn, ib, si).start()
                out_copy(k, rb, sw).start()

        if rem:
            k = full - 1
            idx_copy(k, ib0, si0).wait()
            out_copy(k, rb0, sw0).wait()
            pltpu.sync_copy(p_hbm.at[ib0], rb0)
            pltpu.sync_copy(rb0, g_hbm.at[pl.ds(base_o + k * CH, CH), :])
            idx_copy(k, ib1, si1).wait()
            out_copy(k, rb1, sw1).wait()
        else:
            idx_copy(full - 1, ib0, si0).wait()
            idx_copy(full - 1, ib1, si1).wait()
            out_copy(full - 1, rb0, sw0).wait()
            out_copy(full - 1, rb1, sw1).wait()

        if TAIL:
            tb_i = base_i + full * CH
            tb_o = base_o + full * CH
            pltpu.sync_copy(i_hbm.at[pl.ds(tb_i, TAIL)],
                            ib0.at[pl.ds(0, TAIL)])
            pltpu.sync_copy(p_hbm.at[ib0.at[pl.ds(0, TAIL)]],
                            rb0.at[pl.ds(0, TAIL), :])
            pltpu.sync_copy(rb0.at[pl.ds(0, TAIL), :],
                            g_hbm.at[pl.ds(tb_o, TAIL), :])

    return _gather


def _make_scatter(mesh, N, HN, base_rows, n_rows):
    RT, full, rem, TAIL = _sc_chunking(n_rows)
    MAIN = full - rem

    @functools.partial(
        pl.kernel,
        out_type=jax.ShapeDtypeStruct((2, N, HN), jnp.float32),
        mesh=mesh,
        scratch_types=[
            pltpu.VMEM((CH,), jnp.int32), pltpu.VMEM((CH,), jnp.int32),
            pltpu.VMEM((CH, HN), jnp.float32), pltpu.VMEM((CH, HN), jnp.float32),
            pltpu.SemaphoreType.DMA, pltpu.SemaphoreType.DMA,
            pltpu.VMEM((max(8, CH // 16),), jnp.int32),
            pltpu.VMEM_SHARED((N, HN), jnp.float32),
        ])
    def _scatter(msg_hbm, i_hbm, zero_hbm, out_hbm,
                 ib0, ib1, mb0, mb1, s0, s1, it_v, acc):
        cid = lax.axis_index("core")
        sid = lax.axis_index("subcore")
        wid = cid * 16 + sid
        base_i = base_rows + wid * RT
        base_m = wid * RT

        @pl.when(sid == 0)
        def _():
            pltpu.sync_copy(zero_hbm, acc)

        plsc.subcore_barrier()

        def in_copies(k, ib, mb, s):
            return (pltpu.make_async_copy(
                        i_hbm.at[pl.ds(base_i + k * CH, CH)], ib, s),
                    pltpu.make_async_copy(
                        msg_hbm.at[pl.ds(base_m + k * CH, CH), :], mb, s))

        for c in in_copies(0, ib0, mb0, s0) + in_copies(1, ib1, mb1, s1):
            c.start()

        @pl.loop(0, MAIN, step=2)
        def _(g):
            for b, (ib, mb, s) in enumerate(((ib0, mb0, s0), (ib1, mb1, s1))):
                k = g + b
                ca, cb = in_copies(k, ib, mb, s)
                ca.wait()
                cb.wait()
                pltpu.sync_copy(mb, acc.at[ib], add=True)
                kn = jnp.minimum(k + 2, full - 1)
                for c in in_copies(kn, ib, mb, s):
                    c.start()

        if rem:
            k = full - 1
            ca, cb = in_copies(k, ib0, mb0, s0)
            ca.wait()
            cb.wait()
            pltpu.sync_copy(mb0, acc.at[ib0], add=True)
            for c in in_copies(k, ib1, mb1, s1):
                c.wait()
        else:
            for c in in_copies(full - 1, ib0, mb0, s0) + in_copies(
                    full - 1, ib1, mb1, s1):
                c.wait()

        if TAIL:
            tb_i = base_i + full * CH
            tb_m = base_m + full * CH
            pltpu.sync_copy(i_hbm.at[pl.ds(tb_i, TAIL)],
                            it_v.at[pl.ds(0, TAIL)])
            pltpu.sync_copy(msg_hbm.at[pl.ds(tb_m, TAIL), :],
                            mb0.at[pl.ds(0, TAIL), :])
            pltpu.sync_copy(mb0.at[pl.ds(0, TAIL), :],
                            acc.at[it_v.at[pl.ds(0, TAIL)]], add=True)

        plsc.subcore_barrier()

        @pl.when(sid == 0)
        def _():
            pltpu.sync_copy(acc, out_hbm.at[cid])

    return _scatter


def kernel(node_emb, edge_emb, i, W1, b1, gamma1, beta1, gamma2, beta2):
    N, HN = node_emb.shape
    E, HE = edge_emb.shape
    H2 = W1.shape[0]

    Wn = W1[:, :HN]
    We = W1[:, HN:]
    prm1 = jnp.concatenate(
        [b1[None], gamma1[None], beta1[None],
         jnp.zeros((5, H2), jnp.float32)], axis=0)
    prm2 = jnp.concatenate(
        [gamma2[None], beta2[None], jnp.zeros((6, HN), jnp.float32)], axis=0)

    mesh = plsc.VectorSubcoreMesh(core_axis_name="core",
                                  subcore_axis_name="subcore")

    NS = 2
    ES = E // NS
    TILE = 4000
    grid_s = ES // TILE
    zeros_init = jnp.zeros((N, HN), jnp.float32)

    P = pl.pallas_call(
        _node_mm_body,
        out_shape=jax.ShapeDtypeStruct((N, HN), jnp.int32),
    )(node_emb, Wn)

    G = [_make_gather(mesh, HN, s * ES, ES)(P, i) for s in range(NS)]

    c1, stats = [], []
    for s in range(NS):
        c1_s, st_s = pl.pallas_call(
            _edge_mm_stats_body,
            grid=(grid_s,),
            in_specs=[
                pl.BlockSpec((TILE, HE), lambda t, s=s: (t + s * grid_s, 0)),
                pl.BlockSpec((TILE, HN), lambda t: (t, 0)),
                pl.BlockSpec((H2, HE), lambda t: (0, 0)),
                pl.BlockSpec((8, H2), lambda t: (0, 0)),
            ],
            out_specs=[
                pl.BlockSpec((TILE, HN), lambda t: (t, 0)),
                pl.BlockSpec((8, H2), lambda t: (0, 0)),
            ],
            out_shape=[
                jax.ShapeDtypeStruct((ES, HN), jnp.int32),
                jax.ShapeDtypeStruct((8, H2), jnp.float32),
            ],
        )(edge_emb, G[s], We, prm1)
        c1.append(c1_s)
        stats.append(st_s)

    partials = []
    for s in range(NS):
        msg_s = pl.pallas_call(
            functools.partial(_act_body, inv_e=1.0 / E, hn=HN),
            grid=(grid_s,),
            in_specs=[
                pl.BlockSpec((TILE, HN), lambda t: (t, 0)),
                pl.BlockSpec((8, H2), lambda t: (0, 0)),
                pl.BlockSpec((8, H2), lambda t: (0, 0)),
                pl.BlockSpec((8, H2), lambda t: (0, 0)),
            ],
            out_specs=pl.BlockSpec((TILE, HN), lambda t: (t, 0)),
            out_shape=jax.ShapeDtypeStruct((ES, HN), jnp.float32),
        )(c1[s], stats[0], stats[1], prm1)
        partials.append(
            _make_scatter(mesh, N, HN, s * ES, ES)(msg_s, i, zeros_init))

    out = pl.pallas_call(
        functools.partial(_final_body, inv_n=1.0 / N),
        out_shape=jax.ShapeDtypeStruct((N, HN), jnp.float32),
    )(partials[0], partials[1], node_emb, prm2)
    return out
